# Initial kernel scaffold; baseline (speedup 1.0000x reference)
import functools

import jax
import jax.numpy as jnp
from jax import lax
from jax.experimental import pallas as pl
from jax.experimental.pallas import tpu as pltpu
from jax.experimental.pallas import tpu_sc as plsc

EPS_NORM = 1e-06

IDX_CHUNK = 128


@functools.cache
def _make_sc_gather(B, V, D):
    info = plsc.get_sparse_core_info()
    NC, NS = info.num_cores, info.num_subcores
    NW = NC * NS
    assert B % NW == 0
    b_per_w = B // NW
    assert b_per_w % IDX_CHUNK == 0
    n_chunks = b_per_w // IDX_CHUNK
    mesh = plsc.VectorSubcoreMesh(core_axis_name="c", subcore_axis_name="s")

    @functools.partial(
        pl.kernel,
        mesh=mesh,
        out_type=jax.ShapeDtypeStruct((B, D), jnp.float32),
        scratch_types=[
            pltpu.VMEM((n_chunks, IDX_CHUNK), jnp.int32),
            pltpu.VMEM((b_per_w, D), jnp.float32),
            pltpu.SemaphoreType.DMA,
        ],
    )
    def sc_gather(idx_hbm, table_hbm, out_hbm, idx_v, rows_v, sem):
        wid = lax.axis_index("s") * NC + lax.axis_index("c")
        base = wid * b_per_w
        pltpu.sync_copy(
            idx_hbm.at[pl.ds(base, b_per_w)].reshape(n_chunks, IDX_CHUNK), idx_v
        )
        for j in range(n_chunks):
            pltpu.async_copy(
                table_hbm.at[idx_v.at[j]],
                rows_v.at[pl.ds(j * IDX_CHUNK, IDX_CHUNK)],
                sem,
            )
        for j in range(n_chunks):
            pltpu.make_async_copy(
                table_hbm.at[idx_v.at[j]],
                rows_v.at[pl.ds(j * IDX_CHUNK, IDX_CHUNK)],
                sem,
            ).wait()
        pltpu.sync_copy(rows_v, out_hbm.at[pl.ds(base, b_per_w)])

    return sc_gather


def _tc_body(x_ref, scale_ref, w_ref, o_ref):
    x = x_ref[...]
    var = jnp.mean(x * x, axis=-1, keepdims=True)
    y = x * lax.rsqrt(var + EPS_NORM) * scale_ref[...]
    z = lax.dot_general(
        y, w_ref[...], (((1,), (1,)), ((), ())),
        preferred_element_type=jnp.float32,
    )
    var2 = jnp.mean(z * z, axis=-1, keepdims=True)
    o_ref[...] = z * lax.rsqrt(var2 + EPS_NORM)


@functools.cache
def _make_tc_norm_proj(B, D_vis, D_txt, BB=512):
    return pl.pallas_call(
        _tc_body,
        grid=(B // BB,),
        in_specs=[
            pl.BlockSpec((BB, D_vis), lambda i: (i, 0)),
            pl.BlockSpec((1, D_vis), lambda i: (0, 0)),
            pl.BlockSpec((D_txt, D_vis), lambda i: (0, 0)),
        ],
        out_specs=pl.BlockSpec((BB, D_txt), lambda i: (i, 0)),
        out_shape=jax.ShapeDtypeStruct((B, D_txt), jnp.float32),
        compiler_params=pltpu.CompilerParams(
            dimension_semantics=("arbitrary",),
        ),
    )


def kernel(input_ids, table, norm_scale, proj_w):
    B = input_ids.shape[0]
    V, D_vis = table.shape
    D_txt = proj_w.shape[0]
    ids = input_ids.astype(jnp.int32)
    gathered = _make_sc_gather(B, V, D_vis)(ids, table)
    return _make_tc_norm_proj(B, D_vis, D_txt)(
        gathered, norm_scale.reshape(1, D_vis), proj_w
    )


# speedup vs baseline: 2.3398x; 2.3398x over previous
import functools

import jax
import jax.numpy as jnp
from jax import lax
from jax.experimental import pallas as pl
from jax.experimental.pallas import tpu as pltpu
from jax.experimental.pallas import tpu_sc as plsc

EPS_NORM = 1e-06

IDX_CHUNK = 128


@functools.cache
def _make_sc_gather(B, V, D):
    info = plsc.get_sparse_core_info()
    NC, NS = info.num_cores, info.num_subcores
    NW = NC * NS
    assert B % NW == 0
    b_per_w = B // NW
    assert b_per_w % IDX_CHUNK == 0
    n_chunks = b_per_w // IDX_CHUNK
    mesh = plsc.VectorSubcoreMesh(core_axis_name="c", subcore_axis_name="s")

    @functools.partial(
        pl.kernel,
        mesh=mesh,
        out_type=jax.ShapeDtypeStruct((B, D), jnp.float32),
        scratch_types=[
            pltpu.VMEM((n_chunks, IDX_CHUNK), jnp.int32),
            pltpu.VMEM((b_per_w, D), jnp.float32),
            pltpu.SemaphoreType.DMA,
        ],
    )
    def sc_gather(idx_hbm, table_hbm, out_hbm, idx_v, rows_v, sem):
        wid = lax.axis_index("s") * NC + lax.axis_index("c")
        base = wid * b_per_w
        pltpu.sync_copy(idx_hbm.at[pl.ds(wid * n_chunks, n_chunks)], idx_v)
        for j in range(n_chunks):
            pltpu.async_copy(
                table_hbm.at[idx_v.at[j]],
                rows_v.at[pl.ds(j * IDX_CHUNK, IDX_CHUNK)],
                sem,
            )
        for j in range(n_chunks):
            pltpu.make_async_copy(
                table_hbm.at[idx_v.at[j]],
                rows_v.at[pl.ds(j * IDX_CHUNK, IDX_CHUNK)],
                sem,
            ).wait()
        pltpu.sync_copy(rows_v, out_hbm.at[pl.ds(base, b_per_w)])

    return sc_gather


def _tc_body(x_ref, scale_ref, w_ref, o_ref):
    x = x_ref[...]
    var = jnp.mean(x * x, axis=-1, keepdims=True)
    y = x * lax.rsqrt(var + EPS_NORM) * scale_ref[...]
    z = lax.dot_general(
        y, w_ref[...], (((1,), (1,)), ((), ())),
        preferred_element_type=jnp.float32,
    )
    var2 = jnp.mean(z * z, axis=-1, keepdims=True)
    o_ref[...] = z * lax.rsqrt(var2 + EPS_NORM)


@functools.cache
def _make_tc_norm_proj(B, D_vis, D_txt, BB=512):
    return pl.pallas_call(
        _tc_body,
        grid=(B // BB,),
        in_specs=[
            pl.BlockSpec((BB, D_vis), lambda i: (i, 0)),
            pl.BlockSpec((1, D_vis), lambda i: (0, 0)),
            pl.BlockSpec((D_txt, D_vis), lambda i: (0, 0)),
        ],
        out_specs=pl.BlockSpec((BB, D_txt), lambda i: (i, 0)),
        out_shape=jax.ShapeDtypeStruct((B, D_txt), jnp.float32),
        compiler_params=pltpu.CompilerParams(
            dimension_semantics=("arbitrary",),
        ),
    )


def kernel(input_ids, table, norm_scale, proj_w):
    B = input_ids.shape[0]
    V, D_vis = table.shape
    D_txt = proj_w.shape[0]
    ids = input_ids.astype(jnp.int32).reshape(B // IDX_CHUNK, IDX_CHUNK)
    gathered = _make_sc_gather(B, V, D_vis)(ids, table)
    return _make_tc_norm_proj(B, D_vis, D_txt)(
        gathered, norm_scale.reshape(1, D_vis), proj_w
    )


# speedup vs baseline: 2.7225x; 1.1636x over previous
import functools

import jax
import jax.numpy as jnp
from jax import lax
from jax.experimental import pallas as pl
from jax.experimental.pallas import tpu as pltpu
from jax.experimental.pallas import tpu_sc as plsc

EPS_NORM = 1e-06

IDX_CHUNK = 128


@functools.cache
def _make_sc_gather(B, V, D):
    info = plsc.get_sparse_core_info()
    NC, NS = info.num_cores, info.num_subcores
    NW = NC * NS
    assert B % NW == 0
    b_per_w = B // NW
    assert b_per_w % IDX_CHUNK == 0
    n_chunks = b_per_w // IDX_CHUNK
    mesh = plsc.VectorSubcoreMesh(core_axis_name="c", subcore_axis_name="s")

    @functools.partial(
        pl.kernel,
        mesh=mesh,
        out_type=jax.ShapeDtypeStruct((B, D), jnp.float32),
        scratch_types=[
            pltpu.VMEM((n_chunks, IDX_CHUNK), jnp.int32),
            pltpu.VMEM((b_per_w, D), jnp.float32),
            pltpu.SemaphoreType.DMA,
        ],
    )
    def sc_gather(idx_hbm, table_hbm, out_hbm, idx_v, rows_v, sem):
        wid = lax.axis_index("s") * NC + lax.axis_index("c")
        base = wid * b_per_w
        pltpu.sync_copy(idx_hbm.at[pl.ds(wid * n_chunks, n_chunks)], idx_v)
        for j in range(n_chunks):
            pltpu.async_copy(
                table_hbm.at[idx_v.at[j]],
                rows_v.at[pl.ds(j * IDX_CHUNK, IDX_CHUNK)],
                sem,
            )
        for j in range(n_chunks):
            pltpu.make_async_copy(
                table_hbm.at[idx_v.at[j]],
                rows_v.at[pl.ds(j * IDX_CHUNK, IDX_CHUNK)],
                sem,
            ).wait()
        pltpu.sync_copy(rows_v, out_hbm.at[pl.ds(base, b_per_w)])

    return sc_gather


def _tc_body(x_ref, scale_ref, w_ref, o_ref):
    x = x_ref[...]
    var = jnp.mean(x * x, axis=-1, keepdims=True)
    y = x * lax.rsqrt(var + EPS_NORM) * scale_ref[...]
    z = lax.dot_general(
        y, w_ref[...], (((1,), (1,)), ((), ())),
        preferred_element_type=jnp.float32,
    )
    var2 = jnp.mean(z * z, axis=-1, keepdims=True)
    o_ref[...] = z * lax.rsqrt(var2 + EPS_NORM)


@functools.cache
def _make_tc_norm_proj(B, D_vis, D_txt, BB=1024):
    return pl.pallas_call(
        _tc_body,
        grid=(B // BB,),
        in_specs=[
            pl.BlockSpec((BB, D_vis), lambda i: (i, 0)),
            pl.BlockSpec((1, D_vis), lambda i: (0, 0)),
            pl.BlockSpec((D_txt, D_vis), lambda i: (0, 0)),
        ],
        out_specs=pl.BlockSpec((BB, D_txt), lambda i: (i, 0)),
        out_shape=jax.ShapeDtypeStruct((B, D_txt), jnp.float32),
        compiler_params=pltpu.CompilerParams(
            dimension_semantics=("arbitrary",),
        ),
    )


def kernel(input_ids, table, norm_scale, proj_w):
    B = input_ids.shape[0]
    V, D_vis = table.shape
    D_txt = proj_w.shape[0]
    ids = input_ids.astype(jnp.int32).reshape(B // IDX_CHUNK, IDX_CHUNK)
    gathered = _make_sc_gather(B, V, D_vis)(ids, table)
    return _make_tc_norm_proj(B, D_vis, D_txt)(
        gathered, norm_scale.reshape(1, D_vis), proj_w
    )


# speedup vs baseline: 2.9157x; 1.0710x over previous
import functools

import jax
import jax.numpy as jnp
from jax import lax
from jax.experimental import pallas as pl
from jax.experimental.pallas import tpu as pltpu
from jax.experimental.pallas import tpu_sc as plsc

EPS_NORM = 1e-06

IDX_CHUNK = 128


@functools.cache
def _make_sc_gather(B, V, D):
    info = plsc.get_sparse_core_info()
    NC, NS = info.num_cores, info.num_subcores
    NW = NC * NS
    assert B % NW == 0
    b_per_w = B // NW
    assert b_per_w % IDX_CHUNK == 0
    n_chunks = b_per_w // IDX_CHUNK
    mesh = plsc.VectorSubcoreMesh(core_axis_name="c", subcore_axis_name="s")

    @functools.partial(
        pl.kernel,
        mesh=mesh,
        out_type=jax.ShapeDtypeStruct((B, D), jnp.float32),
        scratch_types=[
            pltpu.VMEM((n_chunks, IDX_CHUNK), jnp.int32),
            pltpu.VMEM((b_per_w, D), jnp.float32),
            pltpu.SemaphoreType.DMA,
        ],
    )
    def sc_gather(idx_hbm, table_hbm, out_hbm, idx_v, rows_v, sem):
        wid = lax.axis_index("s") * NC + lax.axis_index("c")
        base = wid * b_per_w
        pltpu.sync_copy(idx_hbm.at[pl.ds(wid * n_chunks, n_chunks)], idx_v)
        for j in range(n_chunks):
            pltpu.async_copy(
                table_hbm.at[idx_v.at[j]],
                rows_v.at[pl.ds(j * IDX_CHUNK, IDX_CHUNK)],
                sem,
            )
        for j in range(n_chunks):
            pltpu.make_async_copy(
                table_hbm.at[idx_v.at[j]],
                rows_v.at[pl.ds(j * IDX_CHUNK, IDX_CHUNK)],
                sem,
            ).wait()
        pltpu.sync_copy(rows_v, out_hbm.at[pl.ds(base, b_per_w)])

    return sc_gather


def _tc_body(x_ref, scale_ref, w_ref, o_ref):
    x = x_ref[...]
    var = jnp.mean(x * x, axis=-1, keepdims=True)
    y = x * lax.rsqrt(var + EPS_NORM) * scale_ref[...]
    z = lax.dot_general(
        y, w_ref[...], (((1,), (1,)), ((), ())),
        preferred_element_type=jnp.float32,
    )
    var2 = jnp.mean(z * z, axis=-1, keepdims=True)
    o_ref[...] = z * lax.rsqrt(var2 + EPS_NORM)


@functools.cache
def _make_tc_norm_proj(B, D_vis, D_txt, BB=2048):
    return pl.pallas_call(
        _tc_body,
        grid=(B // BB,),
        in_specs=[
            pl.BlockSpec((BB, D_vis), lambda i: (i, 0)),
            pl.BlockSpec((1, D_vis), lambda i: (0, 0)),
            pl.BlockSpec((D_txt, D_vis), lambda i: (0, 0)),
        ],
        out_specs=pl.BlockSpec((BB, D_txt), lambda i: (i, 0)),
        out_shape=jax.ShapeDtypeStruct((B, D_txt), jnp.float32),
        compiler_params=pltpu.CompilerParams(
            dimension_semantics=("arbitrary",),
        ),
    )


def kernel(input_ids, table, norm_scale, proj_w):
    B = input_ids.shape[0]
    V, D_vis = table.shape
    D_txt = proj_w.shape[0]
    ids = input_ids.astype(jnp.int32).reshape(B // IDX_CHUNK, IDX_CHUNK)
    gathered = _make_sc_gather(B, V, D_vis)(ids, table)
    return _make_tc_norm_proj(B, D_vis, D_txt)(
        gathered, norm_scale.reshape(1, D_vis), proj_w
    )
